# baseline (speedup 1.0000x reference)
import dataclasses
import functools

import jax
import jax.numpy as jnp
from jax import lax
from jax.experimental import pallas as pl
from jax.experimental.pallas import tpu as pltpu
from jax.experimental.pallas import tpu_sc as plsc

N = 10000
CIN = 128
COUT = 128
KDIM = 27

_NPAD = 10240
_B = KDIM * _NPAD
_NW = 32
_PER_W = _B // _NW
_WIN = 128
_NFULL = 67
_TAIL = _PER_W - _NFULL * _WIN
_RING = 4

_MBLK = 2048


def _sc_gather_pack(data, idx):
    mesh = plsc.VectorSubcoreMesh(core_axis_name="c", subcore_axis_name="s")
    cp = pltpu.CompilerParams()
    if "needs_layout_passes" in pltpu.CompilerParams.__dataclass_fields__:
        cp = dataclasses.replace(cp, needs_layout_passes=False)

    @functools.partial(
        pl.kernel,
        out_type=jax.ShapeDtypeStruct((_B // 2, CIN), jnp.int32),
        mesh=mesh,
        compiler_params=cp,
        scratch_types=[
            pltpu.VMEM((_PER_W,), jnp.int32),
            pltpu.VMEM((_RING, _WIN, CIN), jnp.float32),
            pltpu.VMEM((2, _WIN, CIN), jnp.int32),
            pltpu.SemaphoreType.DMA((_RING,)),
            pltpu.SemaphoreType.DMA((2,)),
            pltpu.SemaphoreType.DMA,
        ],
    )
    def gather_kernel(data_hbm, idx_hbm, out_hbm, idx_v, rows_v, pk_v,
                      gsem, wsem, isem):
        wid = lax.axis_index("c") * 16 + lax.axis_index("s")
        base = wid * _PER_W
        pbase = wid * (_PER_W // 2)
        pltpu.async_copy(idx_hbm.at[pl.ds(base, _PER_W)], idx_v, isem).wait()

        def g_start(w, b, n):
            pltpu.make_async_copy(
                data_hbm.at[idx_v.at[pl.ds(w * _WIN, n)]],
                rows_v.at[b, pl.ds(0, n)], gsem.at[b]).start()

        def g_wait(b, n):
            pltpu.make_async_copy(
                data_hbm.at[idx_v.at[pl.ds(0, n)]],
                rows_v.at[b, pl.ds(0, n)], gsem.at[b]).wait()

        def w_start(w, p, n):
            pltpu.make_async_copy(
                pk_v.at[p, pl.ds(0, n)],
                out_hbm.at[pl.ds(pbase + w * (_WIN // 2), n)],
                wsem.at[p]).start()

        def w_wait(p, n):
            pltpu.make_async_copy(
                pk_v.at[p, pl.ds(0, n)],
                out_hbm.at[pl.ds(pbase, n)],
                wsem.at[p]).wait()

        def pack(b, p, half, npairs):
            @pl.loop(0, npairs)
            def _(r2):
                for j in range(0, CIN, 16):
                    pair = plsc.pack(rows_v[b, 2 * r2, pl.ds(j, 16)],
                                     rows_v[b, 2 * r2 + 1, pl.ds(j, 16)],
                                     format=plsc.PackFormat.INTERLEAVED)
                    pk_v[p, half * (_WIN // 2) + r2, pl.ds(j, 16)] = (
                        plsc.bitcast(pair, jnp.int32))

        for b in range(_RING):
            g_start(b, b, _WIN)

        for b in range(_RING):
            g_wait(b, _WIN)
            pack(b, b // 2, b % 2, _WIN // 2)
            if b % 2 == 1:
                w_start(b - 1, b // 2, _WIN)
            g_start(_RING + b, b, _WIN)

        @pl.loop(_RING, _NFULL - 2 * _RING + 1, step=_RING)
        def _(w):
            for b in range(_RING):
                g_wait(b, _WIN)
                if b % 2 == 0:
                    w_wait(b // 2, _WIN)
                pack(b, b // 2, b % 2, _WIN // 2)
                if b % 2 == 1:
                    w_start(w + b - 1, b // 2, _WIN)
                g_start(w + _RING + b, b, _WIN)

        for b, w in enumerate(range(60, 64)):
            g_wait(b, _WIN)
            if b % 2 == 0:
                w_wait(b // 2, _WIN)
            pack(b, b // 2, b % 2, _WIN // 2)
            if b % 2 == 1:
                w_start(w - 1, b // 2, _WIN)
            g_start(w + _RING, b, _WIN if w + _RING < _NFULL else _TAIL)

        for b, w in enumerate(range(64, 68)):
            n = _WIN if w < _NFULL else _TAIL
            g_wait(b, n)
            if b % 2 == 0:
                w_wait(b // 2, _WIN)
            pack(b, b // 2, b % 2, n // 2)
            if b % 2 == 1:
                w_start(w - 1, b // 2, (_WIN + n) // 2)

        w_wait(0, _WIN)
        w_wait(1, (_WIN + _TAIL) // 2)

    return gather_kernel(data, idx)


def _gemm_body(buf_ref, w_ref, out_ref):
    acc = jnp.zeros_like(out_ref)
    for k in range(KDIM):
        a = pltpu.bitcast(buf_ref[k], jnp.bfloat16)
        w = w_ref[k].astype(jnp.bfloat16)
        acc += jnp.dot(a, w, preferred_element_type=jnp.float32)
    out_ref[...] = acc


def _tc_gemm(packed, weights):
    n_m = _NPAD // _MBLK
    buffer3 = packed.reshape(KDIM, _NPAD // 2, CIN)
    return pl.pallas_call(
        _gemm_body,
        grid=(n_m,),
        in_specs=[
            pl.BlockSpec((KDIM, _MBLK // 2, CIN), lambda m: (0, m, 0)),
            pl.BlockSpec((KDIM, CIN, COUT), lambda m: (0, 0, 0)),
        ],
        out_specs=pl.BlockSpec((_MBLK, COUT), lambda m: (m, 0)),
        out_shape=jax.ShapeDtypeStruct((_NPAD, COUT), jnp.float32),
    )(buffer3, weights)


def kernel(data, weights, neigh):
    idx = jnp.pad(neigh.T, ((0, 0), (0, _NPAD - N)))
    idx = idx.reshape(_B)
    packed = _sc_gather_pack(data, idx)
    return _tc_gemm(packed, weights)[:N]
